# Initial kernel scaffold; baseline (speedup 1.0000x reference)
import jax
import jax.numpy as jnp
from jax.experimental import pallas as pl


def kernel(membership, table):
    raise NotImplementedError("write your pallas kernel here")


# speedup vs baseline: 5.1188x; 5.1188x over previous
import dataclasses

import jax
import jax.numpy as jnp
from jax import lax
from jax.experimental import pallas as pl
from jax.experimental.pallas import tpu as pltpu
from jax.experimental.pallas import tpu_sc as plsc

_PACK = 4
_WINDOW = 128
_LANES = 16


def kernel(membership, table):
    B, S, D = membership.shape
    E = table.shape[1]
    N = B * S * D
    G = N // _PACK
    R = _PACK * E
    m_per_step = _WINDOW * _PACK

    codes = jnp.arange(1 << _PACK, dtype=jnp.int32)
    bits = jnp.stack([(codes >> k) & 1 for k in range(_PACK)], axis=1)
    lut = table[bits].reshape(1 << _PACK, R)

    idx = membership.astype(jnp.int32).reshape(1, N)

    mesh = plsc.VectorSubcoreMesh(core_axis_name="core",
                                  subcore_axis_name="subcore")
    cp = pltpu.CompilerParams()
    if "needs_layout_passes" in pltpu.CompilerParams.__dataclass_fields__:
        cp = dataclasses.replace(cp, needs_layout_passes=False)

    @pl.kernel(out_type=jax.ShapeDtypeStruct((G, R), table.dtype), mesh=mesh,
               compiler_params=cp,
               scratch_types=[pltpu.VMEM((_WINDOW,), jnp.int32)])
    def gather_kernel(lut_hbm, idx_hbm, out_hbm, codes_vmem):
        def body(i_vmem, o_vmem):
            zero = jnp.zeros((_LANES,), jnp.int32)
            stride = lax.iota(jnp.int32, _LANES) * _PACK
            for j in range(_WINDOW // _LANES):
                base = j * _LANES * _PACK
                acc = plsc.load_gather(i_vmem, [zero, stride + base])
                for k in range(1, _PACK):
                    v = plsc.load_gather(i_vmem, [zero, stride + (base + k)])
                    acc = acc + (v << k)
                codes_vmem[pl.ds(j * _LANES, _LANES)] = acc
            pltpu.sync_copy(lut_hbm.at[codes_vmem], o_vmem)

        pltpu.emit_pipeline(
            body,
            grid=(N // m_per_step,),
            in_specs=[pl.BlockSpec((1, m_per_step),
                                   index_map=lambda i: (0, i))],
            out_specs=[pl.BlockSpec((_WINDOW, R), index_map=lambda i: (i, 0))],
            core_axis_name=("core", "subcore"),
            dimension_semantics=(pltpu.PARALLEL,),
        )(idx_hbm, out_hbm)

    out = gather_kernel(lut, idx)
    return out.reshape(B, S, D, E)


# speedup vs baseline: 5.1246x; 1.0011x over previous
import dataclasses

import jax
import jax.numpy as jnp
from jax import lax
from jax.experimental import pallas as pl
from jax.experimental.pallas import tpu as pltpu
from jax.experimental.pallas import tpu_sc as plsc

_PACK = 4
_SUBWIN = 128
_NSUB = 2
_WINDOW = _SUBWIN * _NSUB
_LANES = 16


def kernel(membership, table):
    B, S, D = membership.shape
    E = table.shape[1]
    N = B * S * D
    G = N // _PACK
    R = _PACK * E
    m_per_step = _WINDOW * _PACK

    codes = jnp.arange(1 << _PACK, dtype=jnp.int32)
    bits = jnp.stack([(codes >> k) & 1 for k in range(_PACK)], axis=1)
    lut = table[bits].reshape(1 << _PACK, R)

    idx = membership.astype(jnp.int32).reshape(1, N)

    mesh = plsc.VectorSubcoreMesh(core_axis_name="core",
                                  subcore_axis_name="subcore")
    cp = pltpu.CompilerParams()
    if "needs_layout_passes" in pltpu.CompilerParams.__dataclass_fields__:
        cp = dataclasses.replace(cp, needs_layout_passes=False)

    @pl.kernel(out_type=jax.ShapeDtypeStruct((G, R), table.dtype), mesh=mesh,
               compiler_params=cp,
               scratch_types=[pltpu.VMEM((_NSUB, _SUBWIN), jnp.int32),
                              pltpu.SemaphoreType.DMA])
    def gather_kernel(lut_hbm, idx_hbm, out_hbm, codes_vmem, gsem):
        def body(i_vmem, o_vmem):
            zero = jnp.zeros((_LANES,), jnp.int32)
            stride = lax.iota(jnp.int32, _LANES) * _PACK
            for j in range(_WINDOW // _LANES):
                base = j * _LANES * _PACK
                acc = plsc.load_gather(i_vmem, [zero, stride + base])
                for k in range(1, _PACK):
                    v = plsc.load_gather(i_vmem, [zero, stride + (base + k)])
                    acc = acc + (v << k)
                codes_vmem[j // (_SUBWIN // _LANES),
                           pl.ds((j % (_SUBWIN // _LANES)) * _LANES, _LANES)] = acc
            copies = [
                pltpu.async_copy(lut_hbm.at[codes_vmem.at[s]],
                                 o_vmem.at[pl.ds(s * _SUBWIN, _SUBWIN)], gsem)
                for s in range(_NSUB)
            ]
            for c in copies:
                c.wait()

        pltpu.emit_pipeline(
            body,
            grid=(N // m_per_step,),
            in_specs=[pl.BlockSpec((1, m_per_step),
                                   index_map=lambda i: (0, i))],
            out_specs=[pl.BlockSpec((_WINDOW, R), index_map=lambda i: (i, 0))],
            core_axis_name=("core", "subcore"),
            dimension_semantics=(pltpu.PARALLEL,),
        )(idx_hbm, out_hbm)

    out = gather_kernel(lut, idx)
    return out.reshape(B, S, D, E)


# speedup vs baseline: 9.2424x; 1.8036x over previous
import dataclasses

import jax
import jax.numpy as jnp
from jax import lax
from jax.experimental import pallas as pl
from jax.experimental.pallas import tpu as pltpu
from jax.experimental.pallas import tpu_sc as plsc

_PACK = 4
_SUBWIN = 128
_NSUB = 2
_WINDOW = _SUBWIN * _NSUB
_LANES = 16


def kernel(membership, table):
    B, S, D = membership.shape
    E = table.shape[1]
    N = B * S * D
    G = N // _PACK
    R = _PACK * E
    m_per_step = _WINDOW * _PACK

    codes = jnp.arange(1 << _PACK, dtype=jnp.int32)
    bits = jnp.stack([(codes >> k) & 1 for k in range(_PACK)], axis=1)
    nrows = 1 << _PACK
    lut = jnp.tile(table[bits].reshape(1, nrows, R), (32, 1, 1))
    lut = lut.reshape(32 * nrows, R)

    idx = membership.astype(jnp.int32).reshape(1, N)

    mesh = plsc.VectorSubcoreMesh(core_axis_name="core",
                                  subcore_axis_name="subcore")
    cp = pltpu.CompilerParams()
    if "needs_layout_passes" in pltpu.CompilerParams.__dataclass_fields__:
        cp = dataclasses.replace(cp, needs_layout_passes=False)

    @pl.kernel(out_type=jax.ShapeDtypeStruct((G, R), table.dtype), mesh=mesh,
               compiler_params=cp,
               scratch_types=[pltpu.VMEM((_NSUB, _SUBWIN), jnp.int32),
                              pltpu.SemaphoreType.DMA])
    def gather_kernel(lut_hbm, idx_hbm, out_hbm, codes_vmem, gsem):
        def body(i_vmem, o_vmem):
            wid = (lax.axis_index("subcore") * 2
                   + lax.axis_index("core")).astype(jnp.int32)
            lut_off = wid * (1 << _PACK)
            zero = jnp.zeros((_LANES,), jnp.int32)
            stride = lax.iota(jnp.int32, _LANES) * _PACK
            for j in range(_WINDOW // _LANES):
                base = j * _LANES * _PACK
                acc = plsc.load_gather(i_vmem, [zero, stride + base])
                for k in range(1, _PACK):
                    v = plsc.load_gather(i_vmem, [zero, stride + (base + k)])
                    acc = acc + (v << k)
                acc = acc + lut_off
                codes_vmem[j // (_SUBWIN // _LANES),
                           pl.ds((j % (_SUBWIN // _LANES)) * _LANES, _LANES)] = acc
            copies = [
                pltpu.async_copy(lut_hbm.at[codes_vmem.at[s]],
                                 o_vmem.at[pl.ds(s * _SUBWIN, _SUBWIN)], gsem)
                for s in range(_NSUB)
            ]
            for c in copies:
                c.wait()

        pltpu.emit_pipeline(
            body,
            grid=(N // m_per_step,),
            in_specs=[pl.BlockSpec((1, m_per_step),
                                   index_map=lambda i: (0, i))],
            out_specs=[pl.BlockSpec((_WINDOW, R), index_map=lambda i: (i, 0))],
            core_axis_name=("core", "subcore"),
            dimension_semantics=(pltpu.PARALLEL,),
        )(idx_hbm, out_hbm)

    out = gather_kernel(lut, idx)
    return out.reshape(B, S, D, E)


# speedup vs baseline: 9.4507x; 1.0225x over previous
import dataclasses

import jax
import jax.numpy as jnp
from jax import lax
from jax.experimental import pallas as pl
from jax.experimental.pallas import tpu as pltpu
from jax.experimental.pallas import tpu_sc as plsc

_LANES = 16
_NWORKERS = 32
_BTILE = 128
_STILE = 8


def kernel(membership, table):
    B, S, D = membership.shape
    E = table.shape[1]
    n_stiles = (S + _STILE - 1) // _STILE
    tail_rows = S - (n_stiles - 1) * _STILE
    n_steps = D * n_stiles

    m_phys = membership.astype(jnp.int32).transpose(2, 1, 0)
    t_flat = jnp.concatenate([table[0], table[1]])

    mesh = plsc.VectorSubcoreMesh(core_axis_name="core",
                                  subcore_axis_name="subcore")
    cp = pltpu.CompilerParams()
    if "needs_layout_passes" in pltpu.CompilerParams.__dataclass_fields__:
        cp = dataclasses.replace(cp, needs_layout_passes=False)

    @pl.kernel(out_type=jax.ShapeDtypeStruct((S, D, E, B), table.dtype),
               mesh=mesh, compiler_params=cp,
               scratch_types=[pltpu.VMEM((2 * E,), jnp.float32),
                              pltpu.VMEM((2, _STILE, _BTILE), jnp.int32),
                              pltpu.VMEM((2, _STILE, E, _BTILE), jnp.float32),
                              pltpu.SemaphoreType.DMA((2,)),
                              pltpu.SemaphoreType.DMA((2,))])
    def sc_kernel(m_hbm, t_hbm, out_hbm, t_v, m_v, o_v, msem, osem):
        wid = (lax.axis_index("subcore") * 2
               + lax.axis_index("core")).astype(jnp.int32)
        b0 = wid * _BTILE
        pltpu.sync_copy(t_hbm, t_v)

        def start_m(step, buf):
            d = step // n_stiles
            st = step % n_stiles
            return pltpu.async_copy(
                m_hbm.at[d, pl.ds(st * _STILE, _STILE), pl.ds(b0, _BTILE)],
                m_v.at[buf], msem.at[buf])

        def compute_row(sp, buf):
            for g in range(_BTILE // _LANES):
                m16 = m_v[buf, sp, pl.ds(g * _LANES, _LANES)]
                midx = m16 * E
                for e in range(E):
                    vals = plsc.load_gather(t_v, [midx + e])
                    o_v[buf, sp, e, pl.ds(g * _LANES, _LANES)] = vals

        start_m(0, 0)

        @pl.loop(0, n_steps)
        def _(i):
            buf = lax.rem(i, 2)
            d = i // n_stiles
            st = lax.rem(i, n_stiles)
            pltpu.make_async_copy(
                m_hbm.at[0, pl.ds(0, _STILE), pl.ds(b0, _BTILE)],
                m_v.at[buf], msem.at[buf]).wait()
            nxt = jnp.minimum(i + 1, n_steps - 1)
            start_m(nxt, 1 - buf)

            st_prev = lax.rem(i - 2, n_stiles)

            @pl.when(jnp.logical_and(i >= 2, st_prev != n_stiles - 1))
            def _():
                pltpu.make_async_copy(
                    o_v.at[buf],
                    out_hbm.at[pl.ds(0, _STILE), 0, slice(None),
                               pl.ds(b0, _BTILE)],
                    osem.at[buf]).wait()

            @pl.when(jnp.logical_and(i >= 2, st_prev == n_stiles - 1))
            def _():
                pltpu.make_async_copy(
                    o_v.at[buf, pl.ds(0, tail_rows)],
                    out_hbm.at[pl.ds(0, tail_rows), 0, slice(None),
                               pl.ds(b0, _BTILE)],
                    osem.at[buf]).wait()

            @pl.when(st != n_stiles - 1)
            def _():
                @pl.loop(0, _STILE)
                def _(sp):
                    compute_row(sp, buf)
                pltpu.async_copy(
                    o_v.at[buf],
                    out_hbm.at[pl.ds(st * _STILE, _STILE), d, slice(None),
                               pl.ds(b0, _BTILE)],
                    osem.at[buf])

            @pl.when(st == n_stiles - 1)
            def _():
                @pl.loop(0, tail_rows)
                def _(sp):
                    compute_row(sp, buf)
                pltpu.async_copy(
                    o_v.at[buf, pl.ds(0, tail_rows)],
                    out_hbm.at[pl.ds(st * _STILE, tail_rows), d, slice(None),
                               pl.ds(b0, _BTILE)],
                    osem.at[buf])

        pltpu.make_async_copy(
            m_hbm.at[0, pl.ds(0, _STILE), pl.ds(b0, _BTILE)],
            m_v.at[0], msem.at[0]).wait()
        for buf, step in ((0, n_steps - 2), (1, n_steps - 1)):
            st = step % n_stiles
            rows = tail_rows if st == n_stiles - 1 else _STILE
            pltpu.make_async_copy(
                o_v.at[buf, pl.ds(0, rows)],
                out_hbm.at[pl.ds(0, rows), 0, slice(None), pl.ds(b0, _BTILE)],
                osem.at[buf]).wait()

    out_phys = sc_kernel(m_phys, t_flat)
    return out_phys.transpose(3, 0, 1, 2)


# speedup vs baseline: 91.7362x; 9.7068x over previous
import dataclasses

import jax
import jax.numpy as jnp
from jax import lax
from jax.experimental import pallas as pl
from jax.experimental.pallas import tpu as pltpu
from jax.experimental.pallas import tpu_sc as plsc

_LANES = 16
_NWORKERS = 32
_BTILE = 128
_STILE = 8


def kernel(membership, table):
    B, S, D = membership.shape
    E = table.shape[1]
    n_stiles = (S + _STILE - 1) // _STILE
    tail_rows = S - (n_stiles - 1) * _STILE
    n_steps = D * n_stiles

    m_phys = membership.astype(jnp.int32).transpose(2, 1, 0)
    t_splat = jnp.tile(table[:, :, None], (1, 1, _LANES))

    mesh = plsc.VectorSubcoreMesh(core_axis_name="core",
                                  subcore_axis_name="subcore")
    cp = pltpu.CompilerParams()
    if "needs_layout_passes" in pltpu.CompilerParams.__dataclass_fields__:
        cp = dataclasses.replace(cp, needs_layout_passes=False)

    @pl.kernel(out_type=jax.ShapeDtypeStruct((S, D, E, B), table.dtype),
               mesh=mesh, compiler_params=cp,
               scratch_types=[pltpu.VMEM((2, E, _LANES), jnp.float32),
                              pltpu.VMEM((2, _STILE, _BTILE), jnp.int32),
                              pltpu.VMEM((2, _STILE, E, _BTILE), jnp.float32),
                              pltpu.SemaphoreType.DMA((2,)),
                              pltpu.SemaphoreType.DMA((2,))])
    def sc_kernel(m_hbm, t_hbm, out_hbm, ts_v, m_v, o_v, msem, osem):
        wid = (lax.axis_index("subcore") * 2
               + lax.axis_index("core")).astype(jnp.int32)
        b0 = wid * _BTILE
        pltpu.sync_copy(t_hbm, ts_v)

        def start_m(step, buf):
            d = step // n_stiles
            st = step % n_stiles
            return pltpu.async_copy(
                m_hbm.at[d, pl.ds(st * _STILE, _STILE), pl.ds(b0, _BTILE)],
                m_v.at[buf], msem.at[buf])

        def compute_row(sp, buf):
            masks = [m_v[buf, sp, pl.ds(g * _LANES, _LANES)] != 0
                     for g in range(_BTILE // _LANES)]
            for e in range(E):
                t0e = ts_v[0, e]
                t1e = ts_v[1, e]
                for g in range(_BTILE // _LANES):
                    o_v[buf, sp, e, pl.ds(g * _LANES, _LANES)] = jnp.where(
                        masks[g], t1e, t0e)

        start_m(0, 0)

        @pl.loop(0, n_steps)
        def _(i):
            buf = lax.rem(i, 2)
            d = i // n_stiles
            st = lax.rem(i, n_stiles)
            pltpu.make_async_copy(
                m_hbm.at[0, pl.ds(0, _STILE), pl.ds(b0, _BTILE)],
                m_v.at[buf], msem.at[buf]).wait()
            nxt = jnp.minimum(i + 1, n_steps - 1)
            start_m(nxt, 1 - buf)

            st_prev = lax.rem(i - 2, n_stiles)

            @pl.when(jnp.logical_and(i >= 2, st_prev != n_stiles - 1))
            def _():
                pltpu.make_async_copy(
                    o_v.at[buf],
                    out_hbm.at[pl.ds(0, _STILE), 0, slice(None),
                               pl.ds(b0, _BTILE)],
                    osem.at[buf]).wait()

            @pl.when(jnp.logical_and(i >= 2, st_prev == n_stiles - 1))
            def _():
                pltpu.make_async_copy(
                    o_v.at[buf, pl.ds(0, tail_rows)],
                    out_hbm.at[pl.ds(0, tail_rows), 0, slice(None),
                               pl.ds(b0, _BTILE)],
                    osem.at[buf]).wait()

            @pl.when(st != n_stiles - 1)
            def _():
                @pl.loop(0, _STILE)
                def _(sp):
                    compute_row(sp, buf)
                pltpu.async_copy(
                    o_v.at[buf],
                    out_hbm.at[pl.ds(st * _STILE, _STILE), d, slice(None),
                               pl.ds(b0, _BTILE)],
                    osem.at[buf])

            @pl.when(st == n_stiles - 1)
            def _():
                @pl.loop(0, tail_rows)
                def _(sp):
                    compute_row(sp, buf)
                pltpu.async_copy(
                    o_v.at[buf, pl.ds(0, tail_rows)],
                    out_hbm.at[pl.ds(st * _STILE, tail_rows), d, slice(None),
                               pl.ds(b0, _BTILE)],
                    osem.at[buf])

        pltpu.make_async_copy(
            m_hbm.at[0, pl.ds(0, _STILE), pl.ds(b0, _BTILE)],
            m_v.at[0], msem.at[0]).wait()
        for buf, step in ((0, n_steps - 2), (1, n_steps - 1)):
            st = step % n_stiles
            rows = tail_rows if st == n_stiles - 1 else _STILE
            pltpu.make_async_copy(
                o_v.at[buf, pl.ds(0, rows)],
                out_hbm.at[pl.ds(0, rows), 0, slice(None), pl.ds(b0, _BTILE)],
                osem.at[buf]).wait()

    out_phys = sc_kernel(m_phys, t_splat)
    return out_phys.transpose(3, 0, 1, 2)
